# baseline (speedup 1.0000x reference)
import functools

import jax
import jax.numpy as jnp
from jax import lax
from jax.experimental import pallas as pl
from jax.experimental.pallas import tpu as pltpu
from jax.experimental.pallas import tpu_sc as plsc

_NC = 2
_NS = 16
_L = 16
_K = 72
_NBUF = 5
_G = 64
_PAD_ROWS = 16

_BN_EPS = 1e-5


@functools.cache
def _make_sc_agg(n, d, e):
  nw = _NC * _NS
  epw = e // nw
  nchunk = epw // _K
  nacc = n + _PAD_ROWS
  rows_ps = (n // _NS) // 8 * 8
  tail = n - rows_ps * _NS
  zr = 16
  assert rows_ps % zr == 0 and tail <= zr

  mesh = plsc.VectorSubcoreMesh(
      core_axis_name="c", subcore_axis_name="s",
      num_cores=_NC, num_subcores=_NS)

  @functools.partial(
      pl.kernel,
      out_type=jax.ShapeDtypeStruct((_NC, n, d), jnp.float32),
      mesh=mesh,
      scratch_types=(
          [pltpu.VMEM((_K,), jnp.int32)] * _NBUF +
          [pltpu.VMEM((_K,), jnp.int32)] * _NBUF +
          [pltpu.VMEM((_K, d), jnp.float32)] * _NBUF +
          [pltpu.VMEM((zr, d), jnp.float32)] +
          [pltpu.VMEM_SHARED((nacc, d), jnp.float32)] +
          [pltpu.SemaphoreType.DMA] * (3 * _NBUF)
      ),
  )
  def agg_kernel(h_hbm, src_hbm, dst_hbm, out_hbm, *refs):
    srcb = refs[:_NBUF]
    dstb = refs[_NBUF:2 * _NBUF]
    rows = refs[2 * _NBUF:3 * _NBUF]
    zero_v = refs[3 * _NBUF]
    acc_sh = refs[3 * _NBUF + 1]
    gsems = refs[3 * _NBUF + 2:4 * _NBUF + 2]
    ssems = refs[4 * _NBUF + 2:5 * _NBUF + 2]
    dsems = refs[5 * _NBUF + 2:]
    c = lax.axis_index("c")
    s = lax.axis_index("s")
    wid = s * _NC + c

    def isload(t, b):
      pltpu.async_copy(src_hbm.at[wid, t], srcb[b], ssems[b])

    def iswait(b):
      pltpu.make_async_copy(src_hbm.at[wid, 0], srcb[b], ssems[b]).wait()

    def idload(t, b):
      pltpu.async_copy(dst_hbm.at[wid, t], dstb[b], dsems[b])

    def idwait(b):
      pltpu.make_async_copy(dst_hbm.at[wid, 0], dstb[b], dsems[b]).wait()

    def gather(b):
      pltpu.async_copy(h_hbm.at[srcb[b]], rows[b], gsems[b])

    def gwait(b):
      pltpu.make_async_copy(h_hbm.at[srcb[b]], rows[b], gsems[b]).wait()

    for b in range(_NBUF):
      isload(b, b)
      idload(b, b)
    for b in range(_NBUF):
      iswait(b)
      gather(b)

    zeros16 = jnp.zeros((_L,), jnp.float32)

    def zero_body(i, _):
      r = i // (d // _L)
      col = (i % (d // _L)) * _L
      zero_v[r, pl.ds(col, _L)] = zeros16
      return 0

    lax.fori_loop(0, zr * (d // _L), zero_body, 0)
    for j in range(rows_ps // zr):
      pltpu.sync_copy(zero_v, acc_sh.at[pl.ds(s * rows_ps + j * zr, zr)])

    @pl.when(s == 0)
    def _():
      pltpu.sync_copy(zero_v.at[pl.ds(0, tail)],
                      acc_sh.at[pl.ds(_NS * rows_ps, tail)])

    plsc.subcore_barrier()

    def edge_body(i, _):
      t0 = i * _NBUF
      for b in range(_NBUF):
        t = t0 + b
        gwait(b)

        @pl.when(t + _NBUF < nchunk)
        def _():
          isload(t + _NBUF, b)

        idwait(b)
        pltpu.sync_copy(rows[b], acc_sh.at[dstb[b]], add=True)

        @pl.when(t + _NBUF < nchunk)
        def _():
          idload(t + _NBUF, b)
          iswait(b)
          gather(b)

      return 0

    assert nchunk % _NBUF == 0
    lax.fori_loop(0, nchunk // _NBUF, edge_body, 0)
    plsc.subcore_barrier()

    pltpu.sync_copy(acc_sh.at[pl.ds(s * rows_ps, rows_ps)],
                    out_hbm.at[c, pl.ds(s * rows_ps, rows_ps)])

    @pl.when(s == 0)
    def _():
      pltpu.sync_copy(acc_sh.at[pl.ds(_NS * rows_ps, tail)],
                      out_hbm.at[c, pl.ds(_NS * rows_ps, tail)])

  return agg_kernel


_BLK = 2000


def _mask(b_ref, blk):
  bvals = b_ref[0, 0:1, :]
  gids = lax.broadcasted_iota(jnp.int32, (_G, blk), 0)
  return jnp.where(gids == jnp.broadcast_to(bvals, (_G, blk)), 1.0, 0.0)


def _mlp_core(h_ref, a0_ref, a1_ref, w1_ref, s1_ref, t1_ref,
              w2_ref, s2_ref, t2_ref, so_ref, to_ref):
  z = h_ref[...] + a0_ref[...] + a1_ref[...]
  z1 = jnp.dot(z, w1_ref[...], preferred_element_type=jnp.float32)
  z1 = jnp.maximum(z1 * s1_ref[0:1, :] + t1_ref[0:1, :], 0.0)
  z2 = jnp.dot(z1, w2_ref[...], preferred_element_type=jnp.float32)
  z3 = jnp.maximum(z2 * s2_ref[0:1, :] + t2_ref[0:1, :], 0.0)
  return jnp.maximum(z3 * so_ref[0:1, :] + to_ref[0:1, :], 0.0)


def _mlp_specs(n, d, hdim, blk):
  row = lambda i: (i, 0)
  fixed = lambda i: (0, 0)
  vec = pl.BlockSpec((8, hdim), fixed)
  return [
      pl.BlockSpec((blk, d), row),
      pl.BlockSpec((blk, d), row),
      pl.BlockSpec((blk, d), row),
      pl.BlockSpec((d, hdim), fixed), vec, vec,
      pl.BlockSpec((hdim, hdim), fixed), vec, vec,
      vec, vec,
      pl.BlockSpec((1, 8, blk), lambda i: (i, 0, 0)),
  ]


def _mlp1(h, a0, a1, w1, s1, t1, w2, s2, t2, so, to, batch8):
  n, d = h.shape
  hdim = w2.shape[1]
  blk = _BLK
  grid = n // blk

  def body(h_ref, a0_ref, a1_ref, w1_ref, s1_ref, t1_ref,
           w2_ref, s2_ref, t2_ref, so_ref, to_ref, b_ref,
           o_ref, pool0_ref):
    i = pl.program_id(0)
    o_ref[...] = _mlp_core(h_ref, a0_ref, a1_ref, w1_ref, s1_ref, t1_ref,
                           w2_ref, s2_ref, t2_ref, so_ref, to_ref)
    c0 = jnp.dot(_mask(b_ref, blk), h_ref[...],
                 preferred_element_type=jnp.float32)

    @pl.when(i == 0)
    def _():
      pool0_ref[...] = c0

    @pl.when(i > 0)
    def _():
      pool0_ref[...] += c0

  row = lambda i: (i, 0)
  fixed = lambda i: (0, 0)
  return pl.pallas_call(
      body,
      grid=(grid,),
      in_specs=_mlp_specs(n, d, hdim, blk),
      out_specs=[
          pl.BlockSpec((blk, hdim), row),
          pl.BlockSpec((_G, d), fixed),
      ],
      out_shape=[
          jax.ShapeDtypeStruct((n, hdim), jnp.float32),
          jax.ShapeDtypeStruct((_G, d), jnp.float32),
      ],
  )(h, a0, a1, w1, s1, t1, w2, s2, t2, so, to, batch8)


def _mlp2(h, a0, a1, w1, s1, t1, w2, s2, t2, so, to, batch8,
          pool0, p0, p1, p2, bsum):
  n, d = h.shape
  hdim = w2.shape[1]
  o = p0.shape[1]
  blk = _BLK
  grid = n // blk

  def body(h_ref, a0_ref, a1_ref, w1_ref, s1_ref, t1_ref,
           w2_ref, s2_ref, t2_ref, so_ref, to_ref, b_ref,
           pool0_ref, p0_ref, p1_ref, p2_ref, bs_ref,
           o_ref, score_ref, pl1_ref, pl2_ref):
    i = pl.program_id(0)
    h2_blk = _mlp_core(h_ref, a0_ref, a1_ref, w1_ref, s1_ref, t1_ref,
                       w2_ref, s2_ref, t2_ref, so_ref, to_ref)
    o_ref[...] = h2_blk
    m = _mask(b_ref, blk)
    c1 = jnp.dot(m, h_ref[...], preferred_element_type=jnp.float32)
    c2 = jnp.dot(m, h2_blk, preferred_element_type=jnp.float32)

    @pl.when(i == 0)
    def _():
      pl1_ref[...] = c1
      pl2_ref[...] = c2

    @pl.when(i > 0)
    def _():
      pl1_ref[...] += c1
      pl2_ref[...] += c2

    @pl.when(i == grid - 1)
    def _():
      score_ref[...] = (
          jnp.dot(pool0_ref[...], p0_ref[...],
                  preferred_element_type=jnp.float32)
          + jnp.dot(pl1_ref[...], p1_ref[...],
                    preferred_element_type=jnp.float32)
          + jnp.dot(pl2_ref[...], p2_ref[...],
                    preferred_element_type=jnp.float32)
          + bs_ref[0:1, :])

  row = lambda i: (i, 0)
  fixed = lambda i: (0, 0)
  return pl.pallas_call(
      body,
      grid=(grid,),
      in_specs=_mlp_specs(n, d, hdim, blk) + [
          pl.BlockSpec((_G, d), fixed),
          pl.BlockSpec((d, o), fixed),
          pl.BlockSpec((d, o), fixed),
          pl.BlockSpec((d, o), fixed),
          pl.BlockSpec((8, o), fixed),
      ],
      out_specs=[
          pl.BlockSpec((blk, hdim), row),
          pl.BlockSpec((_G, o), fixed),
          pl.BlockSpec((_G, d), fixed),
          pl.BlockSpec((_G, d), fixed),
      ],
      out_shape=[
          jax.ShapeDtypeStruct((n, hdim), jnp.float32),
          jax.ShapeDtypeStruct((_G, o), jnp.float32),
          jax.ShapeDtypeStruct((_G, d), jnp.float32),
          jax.ShapeDtypeStruct((_G, d), jnp.float32),
      ],
  )(h, a0, a1, w1, s1, t1, w2, s2, t2, so, to, batch8,
    pool0, p0, p1, p2, bsum)


def _rep8(v):
  return jnp.broadcast_to(v[None, :], (8, v.shape[0]))


def kernel(n_feat, edge_index, batch, params):
  n, d = n_feat.shape
  e = edge_index.shape[1]
  nw = _NC * _NS
  unit = nw * _K * _NBUF
  e_pad = -(-e // unit) * unit
  npad = e_pad - e
  pad_src = jnp.arange(npad, dtype=jnp.int32) % n
  pad_dst = n + jnp.arange(npad, dtype=jnp.int32) % _PAD_ROWS
  nchunk = e_pad // nw // _K
  src = jnp.concatenate([edge_index[0], pad_src]).reshape(nw, nchunk, _K)
  dst = jnp.concatenate([edge_index[1], pad_dst]).reshape(nw, nchunk, _K)
  batch8 = jnp.broadcast_to(batch.reshape(5, 1, n // 5)[:, :, :], (5, 8, n // 5))
  c = 1.0 / jnp.sqrt(jnp.float32(1.0 + _BN_EPS))

  agg_fn = _make_sc_agg(n, d, e_pad)

  def fold(p):
    s1 = p['bn1_g'] * c
    t1 = p['b1'] * s1 + p['bn1_b']
    s2 = p['bn_apply_g'] * c
    t2 = p['b2'] * s2 + p['bn_apply_b']
    so = p['bn_out_g'] * c
    to = p['bn_out_b']
    return (p['W1'], _rep8(s1), _rep8(t1), p['W2'], _rep8(s2), _rep8(t2),
            _rep8(so), _rep8(to))

  bsum = (params['pred0']['b'] + params['pred1']['b'] + params['pred2']['b'])

  parts = agg_fn(n_feat, src, dst)
  h1, pool0 = _mlp1(n_feat, parts[0], parts[1], *fold(params['gin0']),
                    batch8)
  parts = agg_fn(h1, src, dst)
  _, score, pooled1, pooled2 = _mlp2(
      h1, parts[0], parts[1], *fold(params['gin1']), batch8,
      pool0, params['pred0']['W'], params['pred1']['W'], params['pred2']['W'],
      _rep8(bsum))
  return (score, pooled1, pooled2)


# speedup vs baseline: 1.0247x; 1.0247x over previous
import functools

import jax
import jax.numpy as jnp
from jax import lax
from jax.experimental import pallas as pl
from jax.experimental.pallas import tpu as pltpu
from jax.experimental.pallas import tpu_sc as plsc

_NC = 2
_NS = 16
_L = 16
_K = 88
_NBUF = 4
_G = 64
_PAD_ROWS = 16

_BN_EPS = 1e-5


@functools.cache
def _make_sc_agg(n, d, e):
  nw = _NC * _NS
  epw = e // nw
  nchunk = epw // _K
  nacc = n + _PAD_ROWS
  rows_ps = (n // _NS) // 8 * 8
  tail = n - rows_ps * _NS
  zr = 24
  assert rows_ps % zr == 0 and tail <= zr

  mesh = plsc.VectorSubcoreMesh(
      core_axis_name="c", subcore_axis_name="s",
      num_cores=_NC, num_subcores=_NS)

  @functools.partial(
      pl.kernel,
      out_type=jax.ShapeDtypeStruct((_NC, n, d), jnp.float32),
      mesh=mesh,
      scratch_types=(
          [pltpu.VMEM((_K,), jnp.int32)] * _NBUF +
          [pltpu.VMEM((_K,), jnp.int32)] * _NBUF +
          [pltpu.VMEM((_K, d), jnp.float32)] * _NBUF +
          [pltpu.VMEM((zr, d), jnp.float32)] +
          [pltpu.VMEM_SHARED((nacc, d), jnp.float32)] +
          [pltpu.SemaphoreType.DMA] * (3 * _NBUF)
      ),
  )
  def agg_kernel(h_hbm, src_hbm, dst_hbm, out_hbm, *refs):
    srcb = refs[:_NBUF]
    dstb = refs[_NBUF:2 * _NBUF]
    rows = refs[2 * _NBUF:3 * _NBUF]
    zero_v = refs[3 * _NBUF]
    acc_sh = refs[3 * _NBUF + 1]
    gsems = refs[3 * _NBUF + 2:4 * _NBUF + 2]
    ssems = refs[4 * _NBUF + 2:5 * _NBUF + 2]
    dsems = refs[5 * _NBUF + 2:]
    c = lax.axis_index("c")
    s = lax.axis_index("s")
    wid = s * _NC + c

    def isload(t, b):
      pltpu.async_copy(src_hbm.at[wid, t], srcb[b], ssems[b])

    def iswait(b):
      pltpu.make_async_copy(src_hbm.at[wid, 0], srcb[b], ssems[b]).wait()

    def idload(t, b):
      pltpu.async_copy(dst_hbm.at[wid, t], dstb[b], dsems[b])

    def idwait(b):
      pltpu.make_async_copy(dst_hbm.at[wid, 0], dstb[b], dsems[b]).wait()

    def gather(b):
      pltpu.async_copy(h_hbm.at[srcb[b]], rows[b], gsems[b])

    def gwait(b):
      pltpu.make_async_copy(h_hbm.at[srcb[b]], rows[b], gsems[b]).wait()

    for b in range(_NBUF):
      isload(b, b)
      idload(b, b)
    for b in range(_NBUF):
      iswait(b)
      gather(b)

    zeros16 = jnp.zeros((_L,), jnp.float32)

    def zero_body(i, _):
      r = i // (d // _L)
      col = (i % (d // _L)) * _L
      zero_v[r, pl.ds(col, _L)] = zeros16
      return 0

    lax.fori_loop(0, zr * (d // _L), zero_body, 0)
    for j in range(rows_ps // zr):
      pltpu.sync_copy(zero_v, acc_sh.at[pl.ds(s * rows_ps + j * zr, zr)])

    @pl.when(s == 0)
    def _():
      pltpu.sync_copy(zero_v.at[pl.ds(0, tail)],
                      acc_sh.at[pl.ds(_NS * rows_ps, tail)])

    plsc.subcore_barrier()

    def edge_body(i, _):
      t0 = i * _NBUF
      for b in range(_NBUF):
        t = t0 + b
        gwait(b)

        @pl.when(t + _NBUF < nchunk)
        def _():
          isload(t + _NBUF, b)

        idwait(b)
        pltpu.sync_copy(rows[b], acc_sh.at[dstb[b]], add=True)

        @pl.when(t + _NBUF < nchunk)
        def _():
          idload(t + _NBUF, b)
          iswait(b)
          gather(b)

      return 0

    assert nchunk % _NBUF == 0
    lax.fori_loop(0, nchunk // _NBUF, edge_body, 0)
    plsc.subcore_barrier()

    pltpu.sync_copy(acc_sh.at[pl.ds(s * rows_ps, rows_ps)],
                    out_hbm.at[c, pl.ds(s * rows_ps, rows_ps)])

    @pl.when(s == 0)
    def _():
      pltpu.sync_copy(acc_sh.at[pl.ds(_NS * rows_ps, tail)],
                      out_hbm.at[c, pl.ds(_NS * rows_ps, tail)])

  return agg_kernel


_BLK = 2000


def _mask(b_ref, blk):
  bvals = b_ref[0, 0:1, :]
  gids = lax.broadcasted_iota(jnp.int32, (_G, blk), 0)
  return jnp.where(gids == jnp.broadcast_to(bvals, (_G, blk)), 1.0, 0.0)


def _mlp_core(h_ref, a0_ref, a1_ref, w1_ref, s1_ref, t1_ref,
              w2_ref, s2_ref, t2_ref, so_ref, to_ref):
  z = h_ref[...] + a0_ref[...] + a1_ref[...]
  z1 = jnp.dot(z, w1_ref[...], preferred_element_type=jnp.float32)
  z1 = jnp.maximum(z1 * s1_ref[0:1, :] + t1_ref[0:1, :], 0.0)
  z2 = jnp.dot(z1, w2_ref[...], preferred_element_type=jnp.float32)
  z3 = jnp.maximum(z2 * s2_ref[0:1, :] + t2_ref[0:1, :], 0.0)
  return jnp.maximum(z3 * so_ref[0:1, :] + to_ref[0:1, :], 0.0)


def _mlp_specs(n, d, hdim, blk):
  row = lambda i: (i, 0)
  fixed = lambda i: (0, 0)
  vec = pl.BlockSpec((8, hdim), fixed)
  return [
      pl.BlockSpec((blk, d), row),
      pl.BlockSpec((blk, d), row),
      pl.BlockSpec((blk, d), row),
      pl.BlockSpec((d, hdim), fixed), vec, vec,
      pl.BlockSpec((hdim, hdim), fixed), vec, vec,
      vec, vec,
      pl.BlockSpec((1, 8, blk), lambda i: (i, 0, 0)),
  ]


def _mlp1(h, a0, a1, w1, s1, t1, w2, s2, t2, so, to, batch8):
  n, d = h.shape
  hdim = w2.shape[1]
  blk = _BLK
  grid = n // blk

  def body(h_ref, a0_ref, a1_ref, w1_ref, s1_ref, t1_ref,
           w2_ref, s2_ref, t2_ref, so_ref, to_ref, b_ref,
           o_ref, pool0_ref):
    i = pl.program_id(0)
    o_ref[...] = _mlp_core(h_ref, a0_ref, a1_ref, w1_ref, s1_ref, t1_ref,
                           w2_ref, s2_ref, t2_ref, so_ref, to_ref)
    c0 = jnp.dot(_mask(b_ref, blk), h_ref[...],
                 preferred_element_type=jnp.float32)

    @pl.when(i == 0)
    def _():
      pool0_ref[...] = c0

    @pl.when(i > 0)
    def _():
      pool0_ref[...] += c0

  row = lambda i: (i, 0)
  fixed = lambda i: (0, 0)
  return pl.pallas_call(
      body,
      grid=(grid,),
      in_specs=_mlp_specs(n, d, hdim, blk),
      out_specs=[
          pl.BlockSpec((blk, hdim), row),
          pl.BlockSpec((_G, d), fixed),
      ],
      out_shape=[
          jax.ShapeDtypeStruct((n, hdim), jnp.float32),
          jax.ShapeDtypeStruct((_G, d), jnp.float32),
      ],
  )(h, a0, a1, w1, s1, t1, w2, s2, t2, so, to, batch8)


def _mlp2(h, a0, a1, w1, s1, t1, w2, s2, t2, so, to, batch8,
          pool0, p0, p1, p2, bsum):
  n, d = h.shape
  hdim = w2.shape[1]
  o = p0.shape[1]
  blk = _BLK
  grid = n // blk

  def body(h_ref, a0_ref, a1_ref, w1_ref, s1_ref, t1_ref,
           w2_ref, s2_ref, t2_ref, so_ref, to_ref, b_ref,
           pool0_ref, p0_ref, p1_ref, p2_ref, bs_ref,
           o_ref, score_ref, pl1_ref, pl2_ref):
    i = pl.program_id(0)
    h2_blk = _mlp_core(h_ref, a0_ref, a1_ref, w1_ref, s1_ref, t1_ref,
                       w2_ref, s2_ref, t2_ref, so_ref, to_ref)
    o_ref[...] = h2_blk
    m = _mask(b_ref, blk)
    c1 = jnp.dot(m, h_ref[...], preferred_element_type=jnp.float32)
    c2 = jnp.dot(m, h2_blk, preferred_element_type=jnp.float32)

    @pl.when(i == 0)
    def _():
      pl1_ref[...] = c1
      pl2_ref[...] = c2

    @pl.when(i > 0)
    def _():
      pl1_ref[...] += c1
      pl2_ref[...] += c2

    @pl.when(i == grid - 1)
    def _():
      score_ref[...] = (
          jnp.dot(pool0_ref[...], p0_ref[...],
                  preferred_element_type=jnp.float32)
          + jnp.dot(pl1_ref[...], p1_ref[...],
                    preferred_element_type=jnp.float32)
          + jnp.dot(pl2_ref[...], p2_ref[...],
                    preferred_element_type=jnp.float32)
          + bs_ref[0:1, :])

  row = lambda i: (i, 0)
  fixed = lambda i: (0, 0)
  return pl.pallas_call(
      body,
      grid=(grid,),
      in_specs=_mlp_specs(n, d, hdim, blk) + [
          pl.BlockSpec((_G, d), fixed),
          pl.BlockSpec((d, o), fixed),
          pl.BlockSpec((d, o), fixed),
          pl.BlockSpec((d, o), fixed),
          pl.BlockSpec((8, o), fixed),
      ],
      out_specs=[
          pl.BlockSpec((blk, hdim), row),
          pl.BlockSpec((_G, o), fixed),
          pl.BlockSpec((_G, d), fixed),
          pl.BlockSpec((_G, d), fixed),
      ],
      out_shape=[
          jax.ShapeDtypeStruct((n, hdim), jnp.float32),
          jax.ShapeDtypeStruct((_G, o), jnp.float32),
          jax.ShapeDtypeStruct((_G, d), jnp.float32),
          jax.ShapeDtypeStruct((_G, d), jnp.float32),
      ],
  )(h, a0, a1, w1, s1, t1, w2, s2, t2, so, to, batch8,
    pool0, p0, p1, p2, bsum)


def _rep8(v):
  return jnp.broadcast_to(v[None, :], (8, v.shape[0]))


def kernel(n_feat, edge_index, batch, params):
  n, d = n_feat.shape
  e = edge_index.shape[1]
  nw = _NC * _NS
  unit = nw * _K * _NBUF
  e_pad = -(-e // unit) * unit
  npad = e_pad - e
  pad_src = jnp.arange(npad, dtype=jnp.int32) % n
  pad_dst = n + jnp.arange(npad, dtype=jnp.int32) % _PAD_ROWS
  nchunk = e_pad // nw // _K
  src = jnp.concatenate([edge_index[0], pad_src]).reshape(nw, nchunk, _K)
  dst = jnp.concatenate([edge_index[1], pad_dst]).reshape(nw, nchunk, _K)
  batch8 = jnp.broadcast_to(batch.reshape(5, 1, n // 5)[:, :, :], (5, 8, n // 5))
  c = 1.0 / jnp.sqrt(jnp.float32(1.0 + _BN_EPS))

  agg_fn = _make_sc_agg(n, d, e_pad)

  def fold(p):
    s1 = p['bn1_g'] * c
    t1 = p['b1'] * s1 + p['bn1_b']
    s2 = p['bn_apply_g'] * c
    t2 = p['b2'] * s2 + p['bn_apply_b']
    so = p['bn_out_g'] * c
    to = p['bn_out_b']
    return (p['W1'], _rep8(s1), _rep8(t1), p['W2'], _rep8(s2), _rep8(t2),
            _rep8(so), _rep8(to))

  bsum = (params['pred0']['b'] + params['pred1']['b'] + params['pred2']['b'])

  parts = agg_fn(n_feat, src, dst)
  h1, pool0 = _mlp1(n_feat, parts[0], parts[1], *fold(params['gin0']),
                    batch8)
  parts = agg_fn(h1, src, dst)
  _, score, pooled1, pooled2 = _mlp2(
      h1, parts[0], parts[1], *fold(params['gin1']), batch8,
      pool0, params['pred0']['W'], params['pred1']['W'], params['pred2']['W'],
      _rep8(bsum))
  return (score, pooled1, pooled2)
